# Initial kernel scaffold; baseline (speedup 1.0000x reference)
import jax
import jax.numpy as jnp
from jax.experimental import pallas as pl


def kernel(suit_indices, rank_indices, card_indices, suit_table, rank_table, card_table):
    raise NotImplementedError("write your pallas kernel here")


# speedup vs baseline: 13.2393x; 13.2393x over previous
import functools

import jax
import jax.numpy as jnp
from jax import lax
from jax.experimental import pallas as pl
from jax.experimental.pallas import tpu as pltpu
from jax.experimental.pallas import tpu_sc as plsc

_NUM_SUITS = 5
_NUM_RANKS = 14
_NUM_CARDS = _NUM_SUITS * _NUM_RANKS
_EMB = 64

_NC = 2
_NS = 16
_NW = _NC * _NS

_C = 512
_IDXW = 128
_NJ = _C // _IDXW

_ROWS = _NUM_SUITS * _NUM_RANKS * _NUM_CARDS
_ROWS_PAD = ((_ROWS + _NS - 1) // _NS + 7) // 8 * 8 * _NS


def _make_sc_lookup(n_tokens: int):
    cpw = n_tokens // _NW
    chunks = cpw // _C
    stage = _ROWS_PAD // _NS

    mesh = plsc.VectorSubcoreMesh(core_axis_name="c", subcore_axis_name="s")

    @functools.partial(
        pl.kernel,
        out_type=jax.ShapeDtypeStruct((n_tokens, _EMB), jnp.float32),
        mesh=mesh,
        scratch_types=[
            pltpu.VMEM((_C,), jnp.int32),
            pltpu.VMEM((_C,), jnp.int32),
            pltpu.VMEM((_C,), jnp.int32),
            pltpu.VMEM((_NJ, _IDXW), jnp.int32),
            pltpu.VMEM((_C, _EMB), jnp.float32),
            pltpu.VMEM_SHARED((_ROWS_PAD, _EMB), jnp.float32),
            pltpu.SemaphoreType.DMA,
        ],
        compiler_params=pltpu.CompilerParams(use_tc_tiling_on_sc=False),
    )
    def sc_lookup(si_hbm, ri_hbm, ci_hbm, comb_hbm, out_hbm,
                  si_v, ri_v, ci_v, cidx_v, rows_v, table_sh, sem):
        cid = lax.axis_index("c")
        sid = lax.axis_index("s")
        wid = sid * _NC + cid

        pltpu.sync_copy(comb_hbm.at[pl.ds(sid * stage, stage)],
                        rows_v.at[pl.ds(0, stage)])
        pltpu.sync_copy(rows_v.at[pl.ds(0, stage)],
                        table_sh.at[pl.ds(sid * stage, stage)])
        plsc.subcore_barrier()

        def chunk(g, carry):
            base = wid * cpw + g * _C
            pltpu.sync_copy(si_hbm.at[pl.ds(base, _C)], si_v)
            pltpu.sync_copy(ri_hbm.at[pl.ds(base, _C)], ri_v)
            pltpu.sync_copy(ci_hbm.at[pl.ds(base, _C)], ci_v)
            for i in range(_C // 16):
                s = si_v[pl.ds(i * 16, 16)]
                r = ri_v[pl.ds(i * 16, 16)]
                c = ci_v[pl.ds(i * 16, 16)]
                cidx_v[i // 8, pl.ds((i % 8) * 16, 16)] = (s * _NUM_RANKS + r) * _NUM_CARDS + c
            copies = [
                pltpu.async_copy(table_sh.at[cidx_v.at[j]],
                                 rows_v.at[pl.ds(j * _IDXW, _IDXW)], sem)
                for j in range(_NJ)
            ]
            for cp in copies:
                cp.wait()
            pltpu.sync_copy(rows_v, out_hbm.at[pl.ds(base, _C)])
            return carry

        lax.fori_loop(0, chunks, chunk, 0)

    return sc_lookup


def kernel(suit_indices, rank_indices, card_indices, suit_table, rank_table, card_table):
    b, l = suit_indices.shape
    n = b * l

    comb = (suit_table[:, None, None, :]
            + rank_table[None, :, None, :]
            + card_table[None, None, :, :]).reshape(_ROWS, _EMB)
    comb = jnp.pad(comb, ((0, _ROWS_PAD - _ROWS), (0, 0)))

    si = suit_indices.reshape(-1).astype(jnp.int32)
    ri = rank_indices.reshape(-1).astype(jnp.int32)
    ci = card_indices.reshape(-1).astype(jnp.int32)

    grain = _NW * _C
    n_pad = (n + grain - 1) // grain * grain
    if n_pad != n:
        pad = n_pad - n
        si = jnp.pad(si, (0, pad))
        ri = jnp.pad(ri, (0, pad))
        ci = jnp.pad(ci, (0, pad))

    out = _make_sc_lookup(n_pad)(si, ri, ci, comb)
    if n_pad != n:
        out = out[:n]
    return out.reshape(b, l, _EMB)


# speedup vs baseline: 16.2245x; 1.2255x over previous
import functools

import jax
import jax.numpy as jnp
from jax import lax
from jax.experimental import pallas as pl
from jax.experimental.pallas import tpu as pltpu
from jax.experimental.pallas import tpu_sc as plsc

_NUM_SUITS = 5
_NUM_RANKS = 14
_NUM_CARDS = _NUM_SUITS * _NUM_RANKS
_EMB = 64

_NC = 2
_NS = 16
_NW = _NC * _NS

_C = 512
_IDXW = 128
_NJ = _C // _IDXW

_ROWS = _NUM_SUITS * _NUM_RANKS * _NUM_CARDS
_ROWS_PAD = ((_ROWS + _NS - 1) // _NS + 7) // 8 * 8 * _NS


def _make_sc_lookup(n_tokens: int):
    cpw = n_tokens // _NW
    chunks = cpw // _C
    stage = _ROWS_PAD // _NS

    mesh = plsc.VectorSubcoreMesh(core_axis_name="c", subcore_axis_name="s")

    @functools.partial(
        pl.kernel,
        out_type=jax.ShapeDtypeStruct((n_tokens, _EMB), jnp.float32),
        mesh=mesh,
        scratch_types=[
            pltpu.VMEM((2, _C), jnp.int32),
            pltpu.VMEM((2, _C), jnp.int32),
            pltpu.VMEM((2, _C), jnp.int32),
            pltpu.VMEM((2, _NJ, _IDXW), jnp.int32),
            pltpu.VMEM((2, _C, _EMB), jnp.float32),
            pltpu.VMEM_SHARED((_ROWS_PAD, _EMB), jnp.float32),
            pltpu.SemaphoreType.DMA,
            pltpu.SemaphoreType.DMA,
            pltpu.SemaphoreType.DMA,
            pltpu.SemaphoreType.DMA,
            pltpu.SemaphoreType.DMA,
            pltpu.SemaphoreType.DMA,
        ],
        compiler_params=pltpu.CompilerParams(use_tc_tiling_on_sc=False),
    )
    def sc_lookup(si_hbm, ri_hbm, ci_hbm, comb_hbm, out_hbm,
                  si_v, ri_v, ci_v, cidx_v, rows_v, table_sh,
                  sem_in0, sem_in1, sem_g0, sem_g1, sem_out0, sem_out1):
        cid = lax.axis_index("c")
        sid = lax.axis_index("s")
        wid = sid * _NC + cid
        sem_in = (sem_in0, sem_in1)
        sem_g = (sem_g0, sem_g1)
        sem_out = (sem_out0, sem_out1)

        pltpu.sync_copy(comb_hbm.at[pl.ds(sid * stage, stage)],
                        rows_v.at[0, pl.ds(0, stage)])
        pltpu.sync_copy(rows_v.at[0, pl.ds(0, stage)],
                        table_sh.at[pl.ds(sid * stage, stage)])
        plsc.subcore_barrier()

        def start_in(g, b):
            base = wid * cpw + g * _C
            pltpu.async_copy(si_hbm.at[pl.ds(base, _C)], si_v.at[b], sem_in[b])
            pltpu.async_copy(ri_hbm.at[pl.ds(base, _C)], ri_v.at[b], sem_in[b])
            pltpu.async_copy(ci_hbm.at[pl.ds(base, _C)], ci_v.at[b], sem_in[b])

        def wait_in(b):
            pltpu.make_async_copy(si_hbm.at[pl.ds(0, _C)], si_v.at[b], sem_in[b]).wait()
            pltpu.make_async_copy(ri_hbm.at[pl.ds(0, _C)], ri_v.at[b], sem_in[b]).wait()
            pltpu.make_async_copy(ci_hbm.at[pl.ds(0, _C)], ci_v.at[b], sem_in[b]).wait()

        def wait_out(b):
            pltpu.make_async_copy(rows_v.at[b], out_hbm.at[pl.ds(0, _C)], sem_out[b]).wait()

        start_in(0, 0)
        start_in(1, 1)

        def half(t, b):
            g = 2 * t + b
            base = wid * cpw + g * _C
            wait_in(b)
            for i in range(_C // 16):
                s = si_v[b, pl.ds(i * 16, 16)]
                r = ri_v[b, pl.ds(i * 16, 16)]
                c = ci_v[b, pl.ds(i * 16, 16)]
                cidx_v[b, i // 8, pl.ds((i % 8) * 16, 16)] = (s * _NUM_RANKS + r) * _NUM_CARDS + c

            @pl.when(g >= 2)
            def _():
                wait_out(b)

            copies = [
                pltpu.async_copy(table_sh.at[cidx_v.at[b, j]],
                                 rows_v.at[b, pl.ds(j * _IDXW, _IDXW)], sem_g[b])
                for j in range(_NJ)
            ]

            @pl.when(g + 2 < chunks)
            def _():
                start_in(g + 2, b)

            for cp in copies:
                cp.wait()
            pltpu.async_copy(rows_v.at[b], out_hbm.at[pl.ds(base, _C)], sem_out[b])

        def outer(t, carry):
            half(t, 0)
            half(t, 1)
            return carry

        lax.fori_loop(0, chunks // 2, outer, 0)
        wait_out(0)
        wait_out(1)

    return sc_lookup


def kernel(suit_indices, rank_indices, card_indices, suit_table, rank_table, card_table):
    b, l = suit_indices.shape
    n = b * l

    comb = (suit_table[:, None, None, :]
            + rank_table[None, :, None, :]
            + card_table[None, None, :, :]).reshape(_ROWS, _EMB)
    comb = jnp.pad(comb, ((0, _ROWS_PAD - _ROWS), (0, 0)))

    si = suit_indices.reshape(-1).astype(jnp.int32)
    ri = rank_indices.reshape(-1).astype(jnp.int32)
    ci = card_indices.reshape(-1).astype(jnp.int32)

    grain = _NW * _C
    n_pad = (n + grain - 1) // grain * grain
    if n_pad != n:
        pad = n_pad - n
        si = jnp.pad(si, (0, pad))
        ri = jnp.pad(ri, (0, pad))
        ci = jnp.pad(ci, (0, pad))

    out = _make_sc_lookup(n_pad)(si, ri, ci, comb)
    if n_pad != n:
        out = out[:n]
    return out.reshape(b, l, _EMB)


# speedup vs baseline: 23.8184x; 1.4681x over previous
import functools

import jax
import jax.numpy as jnp
from jax import lax
from jax.experimental import pallas as pl
from jax.experimental.pallas import tpu as pltpu
from jax.experimental.pallas import tpu_sc as plsc

_NUM_SUITS = 5
_NUM_RANKS = 14
_NUM_CARDS = _NUM_SUITS * _NUM_RANKS
_EMB = 64
_TW = 128

_NC = 2
_NS = 16
_NW = _NC * _NS

_C = 128
_IDXW = 128
_NJ = _C // _IDXW

_ROWS = _NUM_SUITS * _NUM_RANKS * _NUM_CARDS
_ROWS_PAD = ((_ROWS + _NS - 1) // _NS + 7) // 8 * 8 * _NS


def _make_sc_lookup(n_tokens: int):
    cpw = n_tokens // _NW
    chunks = cpw // _C
    stage = _ROWS_PAD // _NS
    hop = stage // 3

    mesh = plsc.VectorSubcoreMesh(core_axis_name="c", subcore_axis_name="s")

    @functools.partial(
        pl.kernel,
        out_type=jax.ShapeDtypeStruct((n_tokens, _EMB), jnp.float32),
        mesh=mesh,
        scratch_types=[
            pltpu.VMEM((2 * _C,), jnp.int32),
            pltpu.VMEM((2 * _C,), jnp.int32),
            pltpu.VMEM((2 * _C,), jnp.int32),
            pltpu.VMEM((2 * _NJ, _IDXW), jnp.int32),
            pltpu.VMEM((2, _C, _TW), jnp.float32),
            pltpu.VMEM((_C, _EMB), jnp.float32),
            pltpu.VMEM_SHARED((_ROWS_PAD, _TW), jnp.float32),
            pltpu.SemaphoreType.DMA,
            pltpu.SemaphoreType.DMA,
            pltpu.SemaphoreType.DMA,
            pltpu.SemaphoreType.DMA,
            pltpu.SemaphoreType.DMA,
        ],
        compiler_params=pltpu.CompilerParams(use_tc_tiling_on_sc=True),
    )
    def sc_lookup(idx_hbm, comb_hbm, out_hbm,
                  si_v, ri_v, ci_v, cidx_v, rows_v, pack_v, table_sh,
                  sem_in0, sem_in1, sem_g0, sem_g1, sem_out):
        cid = lax.axis_index("c")
        sid = lax.axis_index("s")
        wid = sid * _NC + cid
        sem_in = (sem_in0, sem_in1)
        sem_g = (sem_g0, sem_g1)

        for h in range(3):
            pltpu.sync_copy(comb_hbm.at[pl.ds(sid * stage + h * hop, hop)],
                            rows_v.at[0, pl.ds(0, hop)])
            pltpu.sync_copy(rows_v.at[0, pl.ds(0, hop)],
                            table_sh.at[pl.ds(sid * stage + h * hop, hop)])
        plsc.subcore_barrier()

        def start_in(g, b):
            base = wid * cpw + g * _C
            pltpu.async_copy(idx_hbm.at[pl.ds(base, _C)], si_v.at[pl.ds(b * _C, _C)], sem_in[b])
            pltpu.async_copy(idx_hbm.at[pl.ds(n_tokens + base, _C)], ri_v.at[pl.ds(b * _C, _C)], sem_in[b])
            pltpu.async_copy(idx_hbm.at[pl.ds(2 * n_tokens + base, _C)], ci_v.at[pl.ds(b * _C, _C)], sem_in[b])

        def wait_in(b):
            pltpu.make_async_copy(idx_hbm.at[pl.ds(0, _C)], si_v.at[pl.ds(b * _C, _C)], sem_in[b]).wait()
            pltpu.make_async_copy(idx_hbm.at[pl.ds(0, _C)], ri_v.at[pl.ds(b * _C, _C)], sem_in[b]).wait()
            pltpu.make_async_copy(idx_hbm.at[pl.ds(0, _C)], ci_v.at[pl.ds(b * _C, _C)], sem_in[b]).wait()

        def fire_gathers(g, b):
            wait_in(b)
            for i in range(_C // 16):
                off = b * _C + i * 16
                s = si_v[pl.ds(off, 16)]
                r = ri_v[pl.ds(off, 16)]
                c = ci_v[pl.ds(off, 16)]
                cidx_v[b * _NJ + i // 8, pl.ds((i % 8) * 16, 16)] = (
                    (s * _NUM_RANKS + r) * _NUM_CARDS + c)
            for j in range(_NJ):
                pltpu.async_copy(table_sh.at[cidx_v.at[b * _NJ + j]],
                                 rows_v.at[b, pl.ds(j * _IDXW, _IDXW)], sem_g[b])

        def wait_gathers(b):
            for j in range(_NJ):
                pltpu.make_async_copy(table_sh.at[cidx_v.at[b * _NJ + j]],
                                      rows_v.at[b, pl.ds(j * _IDXW, _IDXW)], sem_g[b]).wait()

        def wait_out():
            pltpu.make_async_copy(pack_v, out_hbm.at[pl.ds(0, _C)], sem_out).wait()

        start_in(0, 0)
        start_in(1, 1)
        fire_gathers(0, 0)
        start_in(2, 0)
        fire_gathers(1, 1)
        start_in(3, 1)

        def half(t, b):
            g = 2 * t + b
            wait_gathers(b)

            @pl.when(g >= 1)
            def _():
                wait_out()

            def compact(i8, carry):
                tok = i8 * 8
                for u in range(8):
                    for q in range(4):
                        pack_v[tok + u, pl.ds(q * 16, 16)] = (
                            rows_v[b, tok + u, pl.ds(q * 16, 16)])
                return carry
            lax.fori_loop(0, _C // 8, compact, 0)

            pltpu.async_copy(pack_v, out_hbm.at[pl.ds(wid * cpw + g * _C, _C)], sem_out)

            @pl.when(g + 2 < chunks)
            def _():
                fire_gathers(g + 2, b)

            @pl.when(g + 4 < chunks)
            def _():
                start_in(g + 4, b)

        def outer(t, carry):
            half(t, 0)
            half(t, 1)
            return carry

        lax.fori_loop(0, chunks // 2, outer, 0)
        wait_out()

    return sc_lookup


def kernel(suit_indices, rank_indices, card_indices, suit_table, rank_table, card_table):
    b, l = suit_indices.shape
    n = b * l

    comb = (suit_table[:, None, None, :]
            + rank_table[None, :, None, :]
            + card_table[None, None, :, :]).reshape(_ROWS, _EMB)
    comb = jnp.pad(comb, ((0, _ROWS_PAD - _ROWS), (0, _TW - _EMB)))

    si = suit_indices.reshape(-1).astype(jnp.int32)
    ri = rank_indices.reshape(-1).astype(jnp.int32)
    ci = card_indices.reshape(-1).astype(jnp.int32)

    grain = _NW * _C
    n_pad = (n + grain - 1) // grain * grain
    if n_pad != n:
        pad = n_pad - n
        si = jnp.pad(si, (0, pad))
        ri = jnp.pad(ri, (0, pad))
        ci = jnp.pad(ci, (0, pad))

    idx = jnp.concatenate([si, ri, ci])

    out = _make_sc_lookup(n_pad)(idx, comb)
    if n_pad != n:
        out = out[:n]
    return out.reshape(b, l, _EMB)
